# baseline (speedup 1.0000x reference)
import functools
import jax
import jax.numpy as jnp
from jax import lax
from jax.experimental import pallas as pl
from jax.experimental.pallas import tpu as pltpu
from jax.experimental.pallas import tpu_sc as plsc

S = 2048
B = 64
D = 64
V = 1000000
N = S * B
W = 128
SCALE = 8.0
L = 16

TC = 32000
TH = TC // 2
TGRID = -(-V // TC)

NW = 32
RPW = N // NW
PPW = RPW // B
CHUNKS = RPW // W
NBUF = 2


def _tc_pack(wtT):
    def body(in_ref, out_ref):
        t = in_ref[...].T
        out_ref[:, 0:D] = t[:TH, :]
        out_ref[:, D:2 * D] = t[TH:, :]

    return pl.pallas_call(
        body,
        grid=(TGRID,),
        in_specs=[pl.BlockSpec((D, TC), lambda i: (0, i))],
        out_specs=pl.BlockSpec((TH, 2 * D), lambda i: (i, 0)),
        out_shape=jax.ShapeDtypeStruct((TGRID * TH, 2 * D), jnp.float32),
    )(wtT)


def _sc_embed(tbl, idx, pos):
    mesh = plsc.VectorSubcoreMesh(core_axis_name="core", subcore_axis_name="subcore")

    @functools.partial(
        pl.kernel,
        out_type=jax.ShapeDtypeStruct((S, B, 2 * D), jnp.float32),
        mesh=mesh,
        scratch_types=[
            pltpu.VMEM((RPW,), jnp.int32),
            pltpu.VMEM((PPW, D), jnp.float32),
            pltpu.VMEM((NBUF, W, D), jnp.float32),
            pltpu.VMEM((NBUF, W // B, B, 2 * D), jnp.float32),
            pltpu.SemaphoreType.DMA,
            pltpu.SemaphoreType.DMA,
            pltpu.SemaphoreType.DMA,
            pltpu.SemaphoreType.DMA,
            pltpu.SemaphoreType.DMA,
        ],
        compiler_params=pltpu.CompilerParams(use_tc_tiling_on_sc=False),
    )
    def emb(tbl_hbm, idx_hbm, pos_hbm, out_hbm, idx_v, pos_v, gbuf, obuf,
            gsem0, gsem1, osem0, osem1, psem):
        gsems = (gsem0, gsem1)
        osems = (osem0, osem1)
        wid = lax.axis_index("subcore") * 2 + lax.axis_index("core")
        base = wid * RPW
        pbase = wid * PPW

        pltpu.async_copy(idx_hbm.at[pl.ds(base, RPW)], idx_v, psem).wait()
        pltpu.async_copy(pos_hbm.at[pl.ds(pbase, PPW)], pos_v, psem).wait()

        def gather_copy(g, b):
            return pltpu.make_async_copy(
                tbl_hbm.at[idx_v.at[pl.ds(g * W, W)]], gbuf.at[b], gsems[b]
            )

        sbase = base // B

        def out_copy(g, b):
            return pltpu.make_async_copy(
                obuf.at[b], out_hbm.at[pl.ds(sbase + g * (W // B), W // B)],
                osems[b],
            )

        gather_copy(0, 0).start()

        @pl.loop(0, CHUNKS, step=NBUF)
        def _ring(g0):
            for b in range(NBUF):
                g = g0 + b
                nb = (b + 1) % NBUF

                @pl.when(g + 1 < CHUNKS)
                def _():
                    gather_copy(g + 1, nb).start()

                gather_copy(g, b).wait()

                @pl.when(g >= NBUF)
                def _():
                    out_copy(g - NBUF, b).wait()

                @pl.loop(0, P_PER_STEP := W // B)
                def _pos_group(p):
                    pp = g * (W // B) + p
                    pv0 = pos_v[pp, pl.ds(0 * L, L)]
                    pv1 = pos_v[pp, pl.ds(1 * L, L)]
                    pv2 = pos_v[pp, pl.ds(2 * L, L)]
                    pv3 = pos_v[pp, pl.ds(3 * L, L)]

                    @pl.loop(0, B)
                    def _row(r):
                        row = p * B + r
                        obuf[b, p, r, pl.ds(0 * L, L)] = gbuf[b, row, pl.ds(0 * L, L)] * SCALE + pv0
                        obuf[b, p, r, pl.ds(1 * L, L)] = gbuf[b, row, pl.ds(1 * L, L)] * SCALE + pv1
                        obuf[b, p, r, pl.ds(2 * L, L)] = gbuf[b, row, pl.ds(2 * L, L)] * SCALE + pv2
                        obuf[b, p, r, pl.ds(3 * L, L)] = gbuf[b, row, pl.ds(3 * L, L)] * SCALE + pv3

                out_copy(g, b).start()

        for b in range(NBUF):
            out_copy(CHUNKS - NBUF + b, b).wait()

    return emb(tbl, idx, pos)


def kernel(x, word_table, pos_table):
    tbl = _tc_pack(word_table.T).reshape(TGRID * TC, D)
    xi = x.reshape(N).astype(jnp.int32)
    pidx = 2 * ((xi // TC) * TH + xi % TH) + (xi // TH) % 2
    pos = pos_table[:S]
    return _sc_embed(tbl, pidx, pos)[:, :, :D]


# speedup vs baseline: 1.0558x; 1.0558x over previous
import functools
import jax
import jax.numpy as jnp
from jax import lax
from jax.experimental import pallas as pl
from jax.experimental.pallas import tpu as pltpu
from jax.experimental.pallas import tpu_sc as plsc

S = 2048
B = 64
D = 64
V = 1000000
N = S * B
W = 128
SCALE = 8.0
L = 16

TC = 40960
TH = TC // 2
TGRID = -(-V // TC)

NW = 32
RPW = N // NW
PPW = RPW // B
CHUNKS = RPW // W
NBUF = 2


def _tc_pack(wtT):
    def body(in_ref, out_ref):
        t = in_ref[...].T
        out_ref[:, 0:D] = t[:TH, :]
        out_ref[:, D:2 * D] = t[TH:, :]

    return pl.pallas_call(
        body,
        grid=(TGRID,),
        in_specs=[pl.BlockSpec((D, TC), lambda i: (0, i))],
        out_specs=pl.BlockSpec((TH, 2 * D), lambda i: (i, 0)),
        out_shape=jax.ShapeDtypeStruct((TGRID * TH, 2 * D), jnp.float32),
    )(wtT)


def _sc_embed(tbl, idx, pos):
    mesh = plsc.VectorSubcoreMesh(core_axis_name="core", subcore_axis_name="subcore")

    @functools.partial(
        pl.kernel,
        out_type=jax.ShapeDtypeStruct((S, B, D), jnp.float32),
        mesh=mesh,
        scratch_types=[
            pltpu.VMEM((RPW,), jnp.int32),
            pltpu.VMEM((PPW, D), jnp.float32),
            pltpu.VMEM((NBUF, W, D), jnp.float32),
            pltpu.VMEM((NBUF, W // B, B, D), jnp.float32),
            pltpu.SemaphoreType.DMA,
            pltpu.SemaphoreType.DMA,
            pltpu.SemaphoreType.DMA,
            pltpu.SemaphoreType.DMA,
            pltpu.SemaphoreType.DMA,
        ],
        compiler_params=pltpu.CompilerParams(use_tc_tiling_on_sc=False),
    )
    def emb(tbl_hbm, idx_hbm, pos_hbm, out_hbm, idx_v, pos_v, gbuf, obuf,
            gsem0, gsem1, osem0, osem1, psem):
        gsems = (gsem0, gsem1)
        osems = (osem0, osem1)
        wid = lax.axis_index("subcore") * 2 + lax.axis_index("core")
        base = wid * RPW
        pbase = wid * PPW

        pltpu.async_copy(idx_hbm.at[pl.ds(base, RPW)], idx_v, psem).wait()
        pltpu.async_copy(pos_hbm.at[pl.ds(pbase, PPW)], pos_v, psem).wait()

        def gather_copy(g, b):
            return pltpu.make_async_copy(
                tbl_hbm.at[idx_v.at[pl.ds(g * W, W)]], gbuf.at[b], gsems[b]
            )

        sbase = base // B

        def out_copy(g, b):
            return pltpu.make_async_copy(
                obuf.at[b], out_hbm.at[pl.ds(sbase + g * (W // B), W // B)],
                osems[b],
            )

        gather_copy(0, 0).start()

        @pl.loop(0, CHUNKS, step=NBUF)
        def _ring(g0):
            for b in range(NBUF):
                g = g0 + b
                nb = (b + 1) % NBUF

                @pl.when(g + 1 < CHUNKS)
                def _():
                    gather_copy(g + 1, nb).start()

                gather_copy(g, b).wait()

                @pl.when(g >= NBUF)
                def _():
                    out_copy(g - NBUF, b).wait()

                @pl.loop(0, P_PER_STEP := W // B)
                def _pos_group(p):
                    pp = g * (W // B) + p
                    pv0 = pos_v[pp, pl.ds(0 * L, L)]
                    pv1 = pos_v[pp, pl.ds(1 * L, L)]
                    pv2 = pos_v[pp, pl.ds(2 * L, L)]
                    pv3 = pos_v[pp, pl.ds(3 * L, L)]

                    @pl.loop(0, B)
                    def _row(r):
                        row = p * B + r
                        obuf[b, p, r, pl.ds(0 * L, L)] = gbuf[b, row, pl.ds(0 * L, L)] * SCALE + pv0
                        obuf[b, p, r, pl.ds(1 * L, L)] = gbuf[b, row, pl.ds(1 * L, L)] * SCALE + pv1
                        obuf[b, p, r, pl.ds(2 * L, L)] = gbuf[b, row, pl.ds(2 * L, L)] * SCALE + pv2
                        obuf[b, p, r, pl.ds(3 * L, L)] = gbuf[b, row, pl.ds(3 * L, L)] * SCALE + pv3

                out_copy(g, b).start()

        for b in range(NBUF):
            out_copy(CHUNKS - NBUF + b, b).wait()

    return emb(tbl, idx, pos)


def kernel(x, word_table, pos_table):
    tbl = _tc_pack(word_table.T).reshape(TGRID * TC, D)
    xi = x.reshape(N).astype(jnp.int32)
    pidx = 2 * ((xi // TC) * TH + xi % TH) + (xi // TH) % 2
    pos = pos_table[:S]
    return _sc_embed(tbl, pidx, pos)


# speedup vs baseline: 1.0674x; 1.0110x over previous
import functools
import jax
import jax.numpy as jnp
from jax import lax
from jax.experimental import pallas as pl
from jax.experimental.pallas import tpu as pltpu
from jax.experimental.pallas import tpu_sc as plsc

S = 2048
B = 64
D = 64
V = 1000000
N = S * B
W = 128
SCALE = 8.0
L = 16

TC = 40960
TH = TC // 2
TGRID = -(-V // TC)

NW = 32
RPW = N // NW
PPW = RPW // B
CHUNKS = RPW // W
NBUF = 4


def _tc_pack(wtT):
    def body(in_ref, out_ref):
        t = in_ref[...].T
        out_ref[:, 0:D] = t[:TH, :]
        out_ref[:, D:2 * D] = t[TH:, :]

    return pl.pallas_call(
        body,
        grid=(TGRID,),
        in_specs=[pl.BlockSpec((D, TC), lambda i: (0, i))],
        out_specs=pl.BlockSpec((TH, 2 * D), lambda i: (i, 0)),
        out_shape=jax.ShapeDtypeStruct((TGRID * TH, 2 * D), jnp.float32),
    )(wtT)


def _sc_embed(tbl, idx, pos):
    mesh = plsc.VectorSubcoreMesh(core_axis_name="core", subcore_axis_name="subcore")

    @functools.partial(
        pl.kernel,
        out_type=jax.ShapeDtypeStruct((S, B, D), jnp.float32),
        mesh=mesh,
        scratch_types=[
            pltpu.VMEM((RPW,), jnp.int32),
            pltpu.VMEM((PPW, D), jnp.float32),
            pltpu.VMEM((NBUF, W, D), jnp.float32),
            pltpu.VMEM((NBUF, W // B, B, D), jnp.float32),
            pltpu.SemaphoreType.DMA,
            pltpu.SemaphoreType.DMA,
            pltpu.SemaphoreType.DMA,
            pltpu.SemaphoreType.DMA,
            pltpu.SemaphoreType.DMA,
            pltpu.SemaphoreType.DMA,
            pltpu.SemaphoreType.DMA,
            pltpu.SemaphoreType.DMA,
            pltpu.SemaphoreType.DMA,
        ],
        compiler_params=pltpu.CompilerParams(use_tc_tiling_on_sc=False),
    )
    def emb(tbl_hbm, idx_hbm, pos_hbm, out_hbm, idx_v, pos_v, gbuf, obuf,
            gsem0, gsem1, gsem2, gsem3, osem0, osem1, osem2, osem3, psem):
        gsems = (gsem0, gsem1, gsem2, gsem3)
        osems = (osem0, osem1, osem2, osem3)
        wid = lax.axis_index("subcore") * 2 + lax.axis_index("core")
        base = wid * RPW
        pbase = wid * PPW

        pltpu.async_copy(idx_hbm.at[pl.ds(base, RPW)], idx_v, psem).wait()
        pltpu.async_copy(pos_hbm.at[pl.ds(pbase, PPW)], pos_v, psem).wait()

        def gather_copy(g, b):
            return pltpu.make_async_copy(
                tbl_hbm.at[idx_v.at[pl.ds(g * W, W)]], gbuf.at[b], gsems[b]
            )

        sbase = base // B

        def out_copy(g, b):
            return pltpu.make_async_copy(
                obuf.at[b], out_hbm.at[pl.ds(sbase + g * (W // B), W // B)],
                osems[b],
            )

        gather_copy(0, 0).start()
        gather_copy(1, 1).start()

        @pl.loop(0, CHUNKS, step=NBUF)
        def _ring(g0):
            for b in range(NBUF):
                g = g0 + b
                nb = (b + 1) % NBUF

                @pl.when(g + 2 < CHUNKS)
                def _():
                    gather_copy(g + 2, (b + 2) % NBUF).start()

                gather_copy(g, b).wait()

                @pl.when(g >= NBUF)
                def _():
                    out_copy(g - NBUF, b).wait()

                @pl.loop(0, P_PER_STEP := W // B)
                def _pos_group(p):
                    pp = g * (W // B) + p
                    pv0 = pos_v[pp, pl.ds(0 * L, L)]
                    pv1 = pos_v[pp, pl.ds(1 * L, L)]
                    pv2 = pos_v[pp, pl.ds(2 * L, L)]
                    pv3 = pos_v[pp, pl.ds(3 * L, L)]

                    @pl.loop(0, B)
                    def _row(r):
                        row = p * B + r
                        obuf[b, p, r, pl.ds(0 * L, L)] = gbuf[b, row, pl.ds(0 * L, L)] * SCALE + pv0
                        obuf[b, p, r, pl.ds(1 * L, L)] = gbuf[b, row, pl.ds(1 * L, L)] * SCALE + pv1
                        obuf[b, p, r, pl.ds(2 * L, L)] = gbuf[b, row, pl.ds(2 * L, L)] * SCALE + pv2
                        obuf[b, p, r, pl.ds(3 * L, L)] = gbuf[b, row, pl.ds(3 * L, L)] * SCALE + pv3

                out_copy(g, b).start()

        for b in range(NBUF):
            out_copy(CHUNKS - NBUF + b, b).wait()

    return emb(tbl, idx, pos)


def kernel(x, word_table, pos_table):
    tbl = _tc_pack(word_table.T).reshape(TGRID * TC, D)
    xi = x.reshape(N).astype(jnp.int32)
    pidx = 2 * ((xi // TC) * TH + xi % TH) + (xi // TH) % 2
    pos = pos_table[:S]
    return _sc_embed(tbl, pidx, pos)


# speedup vs baseline: 1.0705x; 1.0030x over previous
import functools
import jax
import jax.numpy as jnp
from jax import lax
from jax.experimental import pallas as pl
from jax.experimental.pallas import tpu as pltpu
from jax.experimental.pallas import tpu_sc as plsc

S = 2048
B = 64
D = 64
V = 1000000
N = S * B
W = 128
SCALE = 8.0
L = 16

TC = 40960
TH = TC // 2
TGRID = -(-V // TC)

NW = 32
RPW = N // NW
PPW = RPW // B
CHUNKS = RPW // W
NBUF = 4


def _tc_pack(wtT):
    def body(in_ref, out_ref):
        t = in_ref[...].T
        out_ref[:, 0:D] = t[:TH, :]
        out_ref[:, D:2 * D] = t[TH:, :]

    return pl.pallas_call(
        body,
        grid=(TGRID,),
        in_specs=[pl.BlockSpec((D, TC), lambda i: (0, i))],
        out_specs=pl.BlockSpec((TH, 2 * D), lambda i: (i, 0)),
        out_shape=jax.ShapeDtypeStruct((TGRID * TH, 2 * D), jnp.float32),
    )(wtT)


def _sc_embed(tbl, idx, pos):
    mesh = plsc.VectorSubcoreMesh(core_axis_name="core", subcore_axis_name="subcore")

    @functools.partial(
        pl.kernel,
        out_type=jax.ShapeDtypeStruct((N, D), jnp.float32),
        mesh=mesh,
        scratch_types=[
            pltpu.VMEM((RPW,), jnp.int32),
            pltpu.VMEM((PPW, D), jnp.float32),
            pltpu.VMEM((NBUF, W, D), jnp.float32),
            pltpu.VMEM((NBUF, W, D), jnp.float32),
            pltpu.SemaphoreType.DMA,
            pltpu.SemaphoreType.DMA,
            pltpu.SemaphoreType.DMA,
            pltpu.SemaphoreType.DMA,
            pltpu.SemaphoreType.DMA,
            pltpu.SemaphoreType.DMA,
            pltpu.SemaphoreType.DMA,
            pltpu.SemaphoreType.DMA,
            pltpu.SemaphoreType.DMA,
        ],
        compiler_params=pltpu.CompilerParams(use_tc_tiling_on_sc=False),
    )
    def emb(tbl_hbm, idx_hbm, pos_hbm, out_hbm, idx_v, pos_v, gbuf, obuf,
            gsem0, gsem1, gsem2, gsem3, osem0, osem1, osem2, osem3, psem):
        gsems = (gsem0, gsem1, gsem2, gsem3)
        osems = (osem0, osem1, osem2, osem3)
        wid = lax.axis_index("subcore") * 2 + lax.axis_index("core")
        base = wid * RPW
        pbase = wid * PPW

        pltpu.async_copy(idx_hbm.at[pl.ds(base, RPW)], idx_v, psem).wait()
        pltpu.async_copy(pos_hbm.at[pl.ds(pbase, PPW)], pos_v, psem).wait()

        def gather_copy(g, b):
            return pltpu.make_async_copy(
                tbl_hbm.at[idx_v.at[pl.ds(g * W, W)]], gbuf.at[b], gsems[b]
            )

        def out_copy(g, b):
            return pltpu.make_async_copy(
                obuf.at[b], out_hbm.at[pl.ds(base + g * W, W)], osems[b]
            )

        gather_copy(0, 0).start()
        gather_copy(1, 1).start()

        @pl.loop(0, CHUNKS, step=NBUF)
        def _ring(g0):
            for b in range(NBUF):
                g = g0 + b
                nb = (b + 1) % NBUF

                @pl.when(g + 2 < CHUNKS)
                def _():
                    gather_copy(g + 2, (b + 2) % NBUF).start()

                gather_copy(g, b).wait()

                @pl.when(g >= NBUF)
                def _():
                    out_copy(g - NBUF, b).wait()

                @pl.loop(0, P_PER_STEP := W // B)
                def _pos_group(p):
                    pp = g * (W // B) + p
                    pv0 = pos_v[pp, pl.ds(0 * L, L)]
                    pv1 = pos_v[pp, pl.ds(1 * L, L)]
                    pv2 = pos_v[pp, pl.ds(2 * L, L)]
                    pv3 = pos_v[pp, pl.ds(3 * L, L)]

                    @pl.loop(0, B)
                    def _row(r):
                        row = p * B + r
                        obuf[b, row, pl.ds(0 * L, L)] = gbuf[b, row, pl.ds(0 * L, L)] * SCALE + pv0
                        obuf[b, row, pl.ds(1 * L, L)] = gbuf[b, row, pl.ds(1 * L, L)] * SCALE + pv1
                        obuf[b, row, pl.ds(2 * L, L)] = gbuf[b, row, pl.ds(2 * L, L)] * SCALE + pv2
                        obuf[b, row, pl.ds(3 * L, L)] = gbuf[b, row, pl.ds(3 * L, L)] * SCALE + pv3

                out_copy(g, b).start()

        for b in range(NBUF):
            out_copy(CHUNKS - NBUF + b, b).wait()

    return emb(tbl, idx, pos)


def kernel(x, word_table, pos_table):
    tbl = _tc_pack(word_table.T).reshape(TGRID * TC, D)
    xi = x.reshape(N).astype(jnp.int32)
    pidx = 2 * ((xi // TC) * TH + xi % TH) + (xi // TH) % 2
    pos = pos_table[:S]
    return _sc_embed(tbl, pidx, pos).reshape(S, B, D)
